# Initial kernel scaffold; baseline (speedup 1.0000x reference)
import jax
import jax.numpy as jnp
from jax.experimental import pallas as pl


def kernel(edge_index, overlap_similarity, overlap_length, W_enc, b_enc, Wa, Wb, Wc, Wd, We, Wd1, bd1, Wd2, bd2):
    raise NotImplementedError("write your pallas kernel here")


# speedup vs baseline: 1.9359x; 1.9359x over previous
import functools

import jax
import jax.numpy as jnp
from jax import lax
from jax.experimental import pallas as pl
from jax.experimental.pallas import tpu as pltpu
from jax.experimental.pallas import tpu_sc as plsc

N = 10000
E = 320000
D = 128
H = 64

NC = 2
NS = 16
NW = NC * NS
EPW = E // NW
C = 80
NCHUNK = EPW // C
RPS = 640
RZ = 80

_SC_MESH = plsc.VectorSubcoreMesh(
    core_axis_name="c", subcore_axis_name="s", num_cores=NC, num_subcores=NS)


def _wid():
    return lax.axis_index("s") * NC + lax.axis_index("c")


def _lane_perm(v, idx):
    return lax.gather(
        v, idx[:, None],
        lax.GatherDimensionNumbers(offset_dims=(), collapsed_slice_dims=(0,),
                                   start_index_map=(0,)),
        (1,), mode=lax.GatherScatterMode.PROMISE_IN_BOUNDS)


def _zero_fill(zbuf, rows, width):
    def fill(i, _):
        for q in range(width // 16):
            zbuf[i, pl.ds(q * 16, 16)] = jnp.zeros((16,), jnp.float32)
        return 0
    lax.fori_loop(0, rows, fill, 0)


def _num_zchunks(sid):
    base = sid * RPS
    return jnp.minimum(RPS, N - base) // RZ


def _zero_spmem(zbuf, acc, sid):
    def z(r, _):
        pltpu.sync_copy(zbuf, acc.at[pl.ds(sid * RPS + r * RZ, RZ), :])
        return 0
    lax.fori_loop(0, _num_zchunks(sid), z, 0)


def _drain_spmem(acc, out_hbm, cid, sid):
    def d(r, _):
        row = sid * RPS + r * RZ
        pltpu.sync_copy(acc.at[pl.ds(row, RZ), :],
                        out_hbm.at[cid, pl.ds(row, RZ), :])
        return 0
    lax.fori_loop(0, _num_zchunks(sid), d, 0)


def _prep_body(olen2, wenc, benc2, wa0, wb0, wd0, we0, wc0, wd1, bd12, wd2row,
               bd2s, out):
    x = olen2[...]
    mean = jnp.sum(x) / E
    var = jnp.sum(x * x) / E - mean * mean
    rstd = lax.rsqrt(var)
    we_ = wenc[...]
    be = benc2[...]
    cw = jnp.dot(we_, wc0[...], preferred_element_type=jnp.float32)
    cb = jnp.dot(be, wc0[...], preferred_element_type=jnp.float32)
    const0 = (jnp.sum(wd0[...], axis=0, keepdims=True)
              + jnp.sum(we0[...], axis=0, keepdims=True) + cb)
    a0 = jnp.sum(wa0[...], axis=0, keepdims=True)
    b0 = jnp.sum(wb0[...], axis=0, keepdims=True)
    w34 = wd1[256:384, :] + wd1[384:512, :]
    u = jnp.dot(we_, w34, preferred_element_type=jnp.float32)
    uc = jnp.dot(be, w34, preferred_element_type=jnp.float32) + bd12[...]
    ones_row = jnp.ones((1, D), jnp.float32)
    out[...] = jnp.concatenate([
        we_[0:1], we_[1:2], be,
        cw[0:1], cw[1:2], const0,
        mean * ones_row, rstd * ones_row,
        a0, b0,
        u[0:1], u[1:2], uc,
        wd2row[...], bd2s[0, 0] * ones_row,
        jnp.zeros((1, D), jnp.float32),
    ], axis=0)


def _prep(olen, W_enc, b_enc, Wa0, Wb0, Wd0, We0, Wc0, Wd1, bd1, Wd2, bd2):
    return pl.pallas_call(
        _prep_body,
        out_shape=jax.ShapeDtypeStruct((16, D), jnp.float32),
    )(olen.reshape(E // D, D), W_enc, b_enc.reshape(1, D), Wa0, Wb0, Wd0, We0,
      Wc0, Wd1, bd1.reshape(1, D), Wd2.reshape(1, D), bd2.reshape(1, 1))


def _l0_body(sim_hbm, olen_hbm, dst_hbm, vecs_hbm,
             e1a_hbm, e1b_hbm, sp_hbm,
             vecs_v, sim_v, olen_v, dst_v, sig_v, ea_v, eb_v, sacc,
             sem0):
    cid = lax.axis_index("c")
    sid = lax.axis_index("s")
    wid = _wid()
    pltpu.sync_copy(vecs_hbm, vecs_v)
    _zero_fill(sig_v, RZ, D)
    _zero_spmem(sig_v, sacc, sid)
    plsc.subcore_barrier()
    mean = vecs_v[6, pl.ds(0, 16)][0]
    rstd = vecs_v[7, pl.ds(0, 16)][0]

    def chunk(j, _):
        base = wid * EPW + j * C
        pltpu.sync_copy(sim_hbm.at[pl.ds(base, C)], sim_v)
        pltpu.sync_copy(olen_hbm.at[pl.ds(base, C)], olen_v)
        pltpu.sync_copy(dst_hbm.at[pl.ds(base, C)], dst_v)

        def group(g, _):
            i0 = g * 16
            sim16 = sim_v[pl.ds(i0, 16)]
            t16 = (olen_v[pl.ds(i0, 16)] - mean) * rstd
            for k in range(16):
                i = i0 + k
                s = sim16[k]
                t = t16[k]
                for q in range(D // 16):
                    sl = pl.ds(q * 16, 16)
                    en = vecs_v[5, sl] + s * vecs_v[3, sl] + t * vecs_v[4, sl]
                    sg = 1.0 / (1.0 + jnp.exp(-en))
                    e1 = (s * vecs_v[0, sl] + t * vecs_v[1, sl]
                          + vecs_v[2, sl] + jnp.maximum(en, 0.0))
                    sig_v[i, sl] = sg
                    if q < H // 16:
                        ea_v[i, pl.ds(q * 16, 16)] = e1
                    else:
                        eb_v[i, pl.ds(q * 16 - H, 16)] = e1
            return 0
        lax.fori_loop(0, C // 16, group, 0)
        pltpu.sync_copy(ea_v, e1a_hbm.at[pl.ds(base, C), :])
        pltpu.sync_copy(eb_v, e1b_hbm.at[pl.ds(base, C), :])
        pltpu.sync_copy(sig_v, sacc.at[dst_v], add=True)
        return 0
    lax.fori_loop(0, NCHUNK, chunk, 0)
    plsc.subcore_barrier()
    _drain_spmem(sacc, sp_hbm, cid, sid)


_l0_call = functools.partial(
    pl.kernel,
    out_type=[
        jax.ShapeDtypeStruct((E, H), jnp.float32),
        jax.ShapeDtypeStruct((E, H), jnp.float32),
        jax.ShapeDtypeStruct((NC, N, D), jnp.float32),
    ],
    mesh=_SC_MESH,
    scratch_types=[
        pltpu.VMEM((16, D), jnp.float32),
        pltpu.VMEM((C,), jnp.float32),
        pltpu.VMEM((C,), jnp.float32),
        pltpu.VMEM((C,), jnp.int32),
        pltpu.VMEM((C, D), jnp.float32),
        pltpu.VMEM((C, H), jnp.float32),
        pltpu.VMEM((C, H), jnp.float32),
        pltpu.VMEM_SHARED((N, D), jnp.float32),
        pltpu.SemaphoreType.DMA,
    ],
)(_l0_body)


def _edge_body(hoff, src_hbm, dst_hbm, db_hbm, eh_hbm, ce_hbm, e_hbm,
               eo_hbm, part_hbm,
               src_v, dst_v, gdb, geh, ce_v, e_v, acc,
               sem1, sem2):
    cid = lax.axis_index("c")
    sid = lax.axis_index("s")
    wid = _wid()
    _zero_fill(gdb, RZ, D)
    _zero_spmem(gdb, acc, sid)
    plsc.subcore_barrier()

    def chunk(j, _):
        base = wid * EPW + j * C
        pltpu.sync_copy(src_hbm.at[pl.ds(base, C)], src_v)
        pltpu.sync_copy(dst_hbm.at[pl.ds(base, C)], dst_v)
        c1 = pltpu.async_copy(db_hbm.at[src_v], gdb, sem1)
        c2 = pltpu.async_copy(eh_hbm.at[dst_v], geh, sem2)
        pltpu.sync_copy(ce_hbm.at[pl.ds(base, C), :], ce_v)
        pltpu.sync_copy(e_hbm.at[pl.ds(base, C), :], e_v)
        c1.wait()
        c2.wait()

        def edge(i, _):
            for q in range(H // 16):
                sl = pl.ds(q * 16, 16)
                en = (gdb[i, sl] + geh[i, pl.ds(hoff + q * 16, 16)]
                      + ce_v[i, sl])
                sg = 1.0 / (1.0 + jnp.exp(-en))
                sb = sg * gdb[i, pl.ds(H + q * 16, 16)]
                geh[i, sl] = sb
                geh[i, pl.ds(H + q * 16, 16)] = sg
                e_v[i, sl] = e_v[i, sl] + jnp.maximum(en, 0.0)
            return 0
        lax.fori_loop(0, C, edge, 0)
        pltpu.sync_copy(e_v, eo_hbm.at[pl.ds(base, C), :])
        pltpu.sync_copy(geh, acc.at[dst_v], add=True)
        return 0
    lax.fori_loop(0, NCHUNK, chunk, 0)
    plsc.subcore_barrier()
    _drain_spmem(acc, part_hbm, cid, sid)


def _make_edge_call(hoff):
    return functools.partial(
        pl.kernel,
        out_type=[
            jax.ShapeDtypeStruct((E, H), jnp.float32),
            jax.ShapeDtypeStruct((NC, N, D), jnp.float32),
        ],
        mesh=_SC_MESH,
        scratch_types=[
            pltpu.VMEM((C,), jnp.int32),
            pltpu.VMEM((C,), jnp.int32),
            pltpu.VMEM((C, D), jnp.float32),
            pltpu.VMEM((C, D), jnp.float32),
            pltpu.VMEM((C, H), jnp.float32),
            pltpu.VMEM((C, H), jnp.float32),
            pltpu.VMEM_SHARED((N, D), jnp.float32),
            pltpu.SemaphoreType.DMA,
            pltpu.SemaphoreType.DMA,
        ],
    )(functools.partial(_edge_body, hoff))


_edge_calls = (_make_edge_call(0), _make_edge_call(H))


def _dec_body(src_hbm, dst_hbm, hs_hbm, hd_hbm, sim_hbm, olen_hbm, vecs_hbm,
              p_hbm,
              vecs_v, src_v, dst_v, sim_v, olen_v, g1, g2, out_v,
              sem1, sem2):
    wid = _wid()
    pltpu.sync_copy(vecs_hbm, vecs_v)
    mean = vecs_v[6, pl.ds(0, 16)][0]
    rstd = vecs_v[7, pl.ds(0, 16)][0]
    bd2 = vecs_v[14, pl.ds(0, 16)][0]
    lane = lax.iota(jnp.int32, 16)

    def chunk(j, _):
        base = wid * EPW + j * C
        pltpu.sync_copy(src_hbm.at[pl.ds(base, C)], src_v)
        pltpu.sync_copy(dst_hbm.at[pl.ds(base, C)], dst_v)
        c1 = pltpu.async_copy(hs_hbm.at[src_v], g1, sem1)
        c2 = pltpu.async_copy(hd_hbm.at[dst_v], g2, sem2)
        pltpu.sync_copy(sim_hbm.at[pl.ds(base, C)], sim_v)
        pltpu.sync_copy(olen_hbm.at[pl.ds(base, C)], olen_v)
        c1.wait()
        c2.wait()

        def group(g, _):
            i0 = g * 16
            sim16 = sim_v[pl.ds(i0, 16)]
            t16 = (olen_v[pl.ds(i0, 16)] - mean) * rstd
            out16 = jnp.zeros((16,), jnp.float32)
            for k in range(16):
                i = i0 + k
                s = sim16[k]
                t = t16[k]
                acc = jnp.zeros((16,), jnp.float32)
                for q in range(D // 16):
                    sl = pl.ds(q * 16, 16)
                    z = (g1[i, sl] + g2[i, sl] + s * vecs_v[10, sl]
                         + t * vecs_v[11, sl] + vecs_v[12, sl])
                    acc = acc + jnp.maximum(z, 0.0) * vecs_v[13, sl]
                for sh in (8, 4, 2, 1):
                    acc = acc + _lane_perm(acc, jnp.bitwise_xor(lane, sh))
                out16 = jnp.where(lane == k, acc, out16)
            out_v[pl.ds(i0, 16)] = out16 + bd2
            return 0
        lax.fori_loop(0, C // 16, group, 0)
        pltpu.sync_copy(out_v, p_hbm.at[pl.ds(base, C)])
        return 0
    lax.fori_loop(0, NCHUNK, chunk, 0)


_dec_call = functools.partial(
    pl.kernel,
    out_type=[jax.ShapeDtypeStruct((E,), jnp.float32)],
    mesh=_SC_MESH,
    scratch_types=[
        pltpu.VMEM((16, D), jnp.float32),
        pltpu.VMEM((C,), jnp.int32),
        pltpu.VMEM((C,), jnp.int32),
        pltpu.VMEM((C,), jnp.float32),
        pltpu.VMEM((C,), jnp.float32),
        pltpu.VMEM((C, D), jnp.float32),
        pltpu.VMEM((C, D), jnp.float32),
        pltpu.VMEM((C,), jnp.float32),
        pltpu.SemaphoreType.DMA,
        pltpu.SemaphoreType.DMA,
    ],
)(_dec_body)


_NBLK = 2000
_EBLK = 4000


def _mm(x, w):
    return jnp.dot(x, w, preferred_element_type=jnp.float32)


def _tables_out(h, wa, wb, wd, we, outs):
    (h_o, ah_o, db0, db1, ehf) = outs
    ah = _mm(h, wa[...])
    bh = _mm(h, wb[...])
    dh = _mm(h, wd[...])
    eh = _mm(h, we[...])
    h_o[...] = h
    ah_o[...] = ah
    db0[...] = jnp.concatenate([dh[:, :H], bh[:, :H]], axis=1)
    db1[...] = jnp.concatenate([dh[:, H:], bh[:, H:]], axis=1)
    ehf[...] = eh


def _hupd(hp, ahp, p0, p1):
    r0 = (p0[0, :, :H] + p0[1, :, :H]) / (p0[0, :, H:] + p0[1, :, H:] + 1e-6)
    r1 = (p1[0, :, :H] + p1[1, :, :H]) / (p1[0, :, H:] + p1[1, :, H:] + 1e-6)
    return hp[...] + jnp.maximum(
        ahp[...] + jnp.concatenate([r0, r1], axis=1), 0.0)


def _node1_body(sp, vecs, wa, wb, wd, we, *outs):
    s = sp[0] + sp[1]
    a0 = vecs[8:9, :]
    b0 = vecs[9:10, :]
    h = 1.0 + jnp.maximum(a0 + (b0 * s) / (s + 1e-6), 0.0)
    _tables_out(h, wa, wb, wd, we, outs)


def _node_body(hp, ahp, p0, p1, wa, wb, wd, we, *outs):
    h = _hupd(hp, ahp, p0, p1)
    _tables_out(h, wa, wb, wd, we, outs)


def _decnode_body(hp, ahp, p0, p1, w1, w2, hs_o, hd_o):
    h = _hupd(hp, ahp, p0, p1)
    hs_o[...] = _mm(h, w1[...])
    hd_o[...] = _mm(h, w2[...])


def _ce_body(e0, e1, wc, ce0, ce1):
    e = jnp.concatenate([e0[...], e1[...]], axis=1)
    ce = _mm(e, wc[...])
    ce0[...] = ce[:, :H]
    ce1[...] = ce[:, H:]


def _nspec(width):
    return pl.BlockSpec((_NBLK, width), lambda i: (i, 0))


def _pspec(width):
    return pl.BlockSpec((NC, _NBLK, width), lambda i: (0, i, 0))


def _wspec(rows=D):
    return pl.BlockSpec((rows, D), lambda i: (0, 0))


_TABLE_OUT_SHAPES = [jax.ShapeDtypeStruct((N, D), jnp.float32)] * 5

_TABLE_OUT_SPECS = [_nspec(D)] * 5


def _node1(sp, vecs, wa, wb, wd, we):
    return pl.pallas_call(
        _node1_body,
        grid=(N // _NBLK,),
        in_specs=[_pspec(D), _wspec(16), _wspec(), _wspec(), _wspec(), _wspec()],
        out_specs=_TABLE_OUT_SPECS,
        out_shape=_TABLE_OUT_SHAPES,
    )(sp, vecs, wa, wb, wd, we)


def _node(hp, ahp, p0, p1, wa, wb, wd, we):
    return pl.pallas_call(
        _node_body,
        grid=(N // _NBLK,),
        in_specs=[_nspec(D), _nspec(D), _pspec(D), _pspec(D),
                  _wspec(), _wspec(), _wspec(), _wspec()],
        out_specs=_TABLE_OUT_SPECS,
        out_shape=_TABLE_OUT_SHAPES,
    )(hp, ahp, p0, p1, wa, wb, wd, we)


def _decnode(hp, ahp, p0, p1, w1, w2):
    return pl.pallas_call(
        _decnode_body,
        grid=(N // _NBLK,),
        in_specs=[_nspec(D), _nspec(D), _pspec(D), _pspec(D),
                  _wspec(), _wspec()],
        out_specs=[_nspec(D), _nspec(D)],
        out_shape=[jax.ShapeDtypeStruct((N, D), jnp.float32),
                   jax.ShapeDtypeStruct((N, D), jnp.float32)],
    )(hp, ahp, p0, p1, w1, w2)


def _ce(e0, e1, wc):
    espec = pl.BlockSpec((_EBLK, H), lambda i: (i, 0))
    return pl.pallas_call(
        _ce_body,
        grid=(E // _EBLK,),
        in_specs=[espec, espec, _wspec()],
        out_specs=[espec, espec],
        out_shape=[jax.ShapeDtypeStruct((E, H), jnp.float32),
                   jax.ShapeDtypeStruct((E, H), jnp.float32)],
    )(e0, e1, wc)


def kernel(edge_index, overlap_similarity, overlap_length, W_enc, b_enc,
           Wa, Wb, Wc, Wd, We, Wd1, bd1, Wd2, bd2):
    src = edge_index[0]
    dst = edge_index[1]
    sim = overlap_similarity
    olen = overlap_length

    vecs = _prep(olen, W_enc, b_enc, Wa[0], Wb[0], Wd[0], We[0], Wc[0],
                 Wd1, bd1, Wd2, bd2)

    e0h, e1h, sp = _l0_call(sim, olen, dst, vecs)

    node_in = _node1(sp, vecs, Wa[1], Wb[1], Wd[1], We[1])
    for l in (1, 2, 3):
        h, ah, db0, db1, ehf = node_in
        ce0, ce1 = _ce(e0h, e1h, Wc[l])
        e0h, p0 = _edge_calls[0](src, dst, db0, ehf, ce0, e0h)
        e1h, p1 = _edge_calls[1](src, dst, db1, ehf, ce1, e1h)
        if l < 3:
            node_in = _node(h, ah, p0, p1,
                            Wa[l + 1], Wb[l + 1], Wd[l + 1], We[l + 1])

    hs, hd = _decnode(h, ah, p0, p1, Wd1[:D], Wd1[D:2 * D])
    (p,) = _dec_call(src, dst, hs, hd, sim, olen, vecs)
    return p.reshape(E, 1)


# speedup vs baseline: 4.1886x; 2.1636x over previous
import functools

import jax
import jax.numpy as jnp
from jax import lax
from jax.experimental import pallas as pl
from jax.experimental.pallas import tpu as pltpu
from jax.experimental.pallas import tpu_sc as plsc

N = 10000
E = 320000
D = 128
H = 64

NC = 2
NS = 16
NW = NC * NS
EPW = E // NW
C = 80
NCHUNK = EPW // C
RPS = 640
RZ = 80

_SC_MESH = plsc.VectorSubcoreMesh(
    core_axis_name="c", subcore_axis_name="s", num_cores=NC, num_subcores=NS)


def _wid():
    return lax.axis_index("s") * NC + lax.axis_index("c")


def _base(j):
    return _wid() * EPW + j * C


def _zero_fill(zbuf, rows, width):
    def fill(i, _):
        for q in range(width // 16):
            zbuf[i, pl.ds(q * 16, 16)] = jnp.zeros((16,), jnp.float32)
        return 0
    lax.fori_loop(0, rows, fill, 0)


def _num_zchunks(sid):
    return jnp.minimum(RPS, N - sid * RPS) // RZ


def _zero_spmem(zbuf, acc, sid):
    def z(r, _):
        pltpu.sync_copy(zbuf, acc.at[pl.ds(sid * RPS + r * RZ, RZ), :])
        return 0
    lax.fori_loop(0, _num_zchunks(sid), z, 0)


def _drain_spmem(acc, out_hbm, cid, sid):
    def d(r, _):
        row = sid * RPS + r * RZ
        pltpu.sync_copy(acc.at[pl.ds(row, RZ), :],
                        out_hbm.at[cid, pl.ds(row, RZ), :])
        return 0
    lax.fori_loop(0, _num_zchunks(sid), d, 0)


def _scat_body(dst_hbm, pack_hbm, part_hbm,
               dst_a, dst_b, pk_a, pk_b, acc,
               sd_a, sd_b, sp_a, sp_b, sc_a, sc_b):
    cid = lax.axis_index("c")
    sid = lax.axis_index("s")
    _zero_fill(pk_a, RZ, D)
    _zero_spmem(pk_a, acc, sid)
    plsc.subcore_barrier()

    def in_start(j, dv, pv, sd, sp):
        pltpu.async_copy(dst_hbm.at[pl.ds(_base(j), C)], dv, sd)
        pltpu.async_copy(pack_hbm.at[pl.ds(_base(j), C), :], pv, sp)

    def in_wait(j, dv, pv, sd, sp):
        pltpu.make_async_copy(dst_hbm.at[pl.ds(_base(j), C)], dv, sd).wait()
        pltpu.make_async_copy(pack_hbm.at[pl.ds(_base(j), C), :], pv, sp).wait()

    def sc_start(dv, pv, sc):
        pltpu.async_copy(pv, acc.at[dv], sc, add=True)

    def sc_wait(dv, pv, sc):
        pltpu.make_async_copy(pv, acc.at[dv], sc).wait()

    in_start(0, dst_a, pk_a, sd_a, sp_a)
    in_start(1, dst_b, pk_b, sd_b, sp_b)

    def pair(j2, _):
        ja = 2 * j2
        jb = ja + 1
        in_wait(ja, dst_a, pk_a, sd_a, sp_a)
        sc_start(dst_a, pk_a, sc_a)
        in_wait(jb, dst_b, pk_b, sd_b, sp_b)
        sc_wait(dst_a, pk_a, sc_a)
        in_start(ja + 2, dst_a, pk_a, sd_a, sp_a)
        sc_start(dst_b, pk_b, sc_b)
        sc_wait(dst_b, pk_b, sc_b)

        @pl.when(jb + 2 < NCHUNK)
        def _():
            in_start(jb + 2, dst_b, pk_b, sd_b, sp_b)
        return 0
    lax.fori_loop(0, NCHUNK // 2, pair, 0)
    in_wait(NCHUNK - 1, dst_a, pk_a, sd_a, sp_a)
    sc_start(dst_a, pk_a, sc_a)
    sc_wait(dst_a, pk_a, sc_a)
    plsc.subcore_barrier()
    _drain_spmem(acc, part_hbm, cid, sid)


_scat = functools.partial(
    pl.kernel,
    out_type=[jax.ShapeDtypeStruct((NC, N, D), jnp.float32)],
    mesh=_SC_MESH,
    scratch_types=[
        pltpu.VMEM((C,), jnp.int32),
        pltpu.VMEM((C,), jnp.int32),
        pltpu.VMEM((C, D), jnp.float32),
        pltpu.VMEM((C, D), jnp.float32),
        pltpu.VMEM_SHARED((N, D), jnp.float32),
    ] + [pltpu.SemaphoreType.DMA] * 6,
)(_scat_body)


def _gather_body(ntab, *refs):
    src_hbm, dst_hbm = refs[0], refs[1]
    tabs = refs[2:2 + ntab]
    outs = refs[2 + ntab:2 + 2 * ntab]
    sc = refs[2 + 2 * ntab:]
    nbuf = 2 + ntab
    seta = sc[:nbuf]
    setb = sc[nbuf:2 * nbuf]
    sems = sc[2 * nbuf:]
    sems_a = sems[:2 + 2 * ntab]
    sems_b = sems[2 + 2 * ntab:]
    idx_of = lambda st: [st[0]] * (ntab - 1) + [st[1]]

    def idx_start(j, st, sm):
        pltpu.async_copy(src_hbm.at[pl.ds(_base(j), C)], st[0], sm[0])
        pltpu.async_copy(dst_hbm.at[pl.ds(_base(j), C)], st[1], sm[1])

    def idx_wait(j, st, sm):
        pltpu.make_async_copy(src_hbm.at[pl.ds(_base(j), C)], st[0], sm[0]).wait()
        pltpu.make_async_copy(dst_hbm.at[pl.ds(_base(j), C)], st[1], sm[1]).wait()

    def g_start(st, sm):
        for k in range(ntab):
            pltpu.async_copy(tabs[k].at[idx_of(st)[k]], st[2 + k], sm[2 + k])

    def g_wait(st, sm):
        for k in range(ntab):
            pltpu.make_async_copy(tabs[k].at[idx_of(st)[k]], st[2 + k],
                                  sm[2 + k]).wait()

    def w_start(j, st, sm):
        for k in range(ntab):
            pltpu.async_copy(st[2 + k], outs[k].at[pl.ds(_base(j), C), :],
                             sm[2 + ntab + k])

    def w_wait(j, st, sm):
        for k in range(ntab):
            pltpu.make_async_copy(st[2 + k],
                                  outs[k].at[pl.ds(_base(j), C), :],
                                  sm[2 + ntab + k]).wait()

    idx_start(0, seta, sems_a)
    idx_start(1, setb, sems_b)

    def pair(j2, _):
        ja = 2 * j2
        jb = ja + 1
        idx_wait(ja, seta, sems_a)

        @pl.when(ja >= 2)
        def _():
            w_wait(ja - 2, seta, sems_a)
        g_start(seta, sems_a)
        idx_wait(jb, setb, sems_b)

        @pl.when(jb >= 2)
        def _():
            w_wait(jb - 2, setb, sems_b)
        g_wait(seta, sems_a)
        w_start(ja, seta, sems_a)
        idx_start(ja + 2, seta, sems_a)
        g_start(setb, sems_b)
        g_wait(setb, sems_b)
        w_start(jb, setb, sems_b)

        @pl.when(jb + 2 < NCHUNK)
        def _():
            idx_start(jb + 2, setb, sems_b)
        return 0
    lax.fori_loop(0, NCHUNK // 2, pair, 0)
    jt = NCHUNK - 1
    idx_wait(jt, seta, sems_a)
    w_wait(jt - 2, seta, sems_a)
    g_start(seta, sems_a)
    g_wait(seta, sems_a)
    w_start(jt, seta, sems_a)
    w_wait(jt, seta, sems_a)
    w_wait(jt - 1, setb, sems_b)


def _make_gather(ntab):
    scratch = []
    for _ in range(2):
        scratch += [pltpu.VMEM((C,), jnp.int32), pltpu.VMEM((C,), jnp.int32)]
        scratch += [pltpu.VMEM((C, D), jnp.float32)] * ntab
    scratch += [pltpu.SemaphoreType.DMA] * (2 * (2 + 2 * ntab))
    return functools.partial(
        pl.kernel,
        out_type=[jax.ShapeDtypeStruct((E, D), jnp.float32)] * ntab,
        mesh=_SC_MESH,
        scratch_types=scratch,
    )(functools.partial(_gather_body, ntab))


_gather3 = _make_gather(3)
_gather2 = _make_gather(2)


def _prep_body(olen2, wenc, benc2, wa0, wb0, wd0, we0, wc0, wd1, bd12, wd2row,
               bd2s, out):
    x = olen2[...]
    mean = jnp.sum(x) / E
    var = jnp.sum(x * x) / E - mean * mean
    rstd = lax.rsqrt(var)
    we_ = wenc[...]
    be = benc2[...]
    cw = jnp.dot(we_, wc0[...], preferred_element_type=jnp.float32)
    cb = jnp.dot(be, wc0[...], preferred_element_type=jnp.float32)
    const0 = (jnp.sum(wd0[...], axis=0, keepdims=True)
              + jnp.sum(we0[...], axis=0, keepdims=True) + cb)
    a0 = jnp.sum(wa0[...], axis=0, keepdims=True)
    b0 = jnp.sum(wb0[...], axis=0, keepdims=True)
    w34 = wd1[256:384, :] + wd1[384:512, :]
    u = jnp.dot(we_, w34, preferred_element_type=jnp.float32)
    uc = jnp.dot(be, w34, preferred_element_type=jnp.float32) + bd12[...]
    ones_row = jnp.ones((1, D), jnp.float32)
    out[...] = jnp.concatenate([
        we_[0:1], we_[1:2], be,
        cw[0:1], cw[1:2], const0,
        mean * ones_row, rstd * ones_row,
        a0, b0,
        u[0:1], u[1:2], uc,
        wd2row[...], bd2s[0, 0] * ones_row,
        jnp.zeros((1, D), jnp.float32),
    ], axis=0)


def _prep(olen, W_enc, b_enc, Wa0, Wb0, Wd0, We0, Wc0, Wd1, bd1, Wd2, bd2):
    return pl.pallas_call(
        _prep_body,
        out_shape=jax.ShapeDtypeStruct((16, D), jnp.float32),
    )(olen.reshape(E // D, D), W_enc, b_enc.reshape(1, D), Wa0, Wb0, Wd0, We0,
      Wc0, Wd1, bd1.reshape(1, D), Wd2.reshape(1, D), bd2.reshape(1, 1))


_NBLK = 2000
_EBLK = 4000


def _mm(x, w):
    return jnp.dot(x, w, preferred_element_type=jnp.float32)


def _st(olen1, v):
    return (olen1[...] - v[6:7, 0:1]) * v[7:8, 0:1]


def _l0tc_body(sim1, olen1, vecs, sig_o, e0_o, e1_o):
    v = vecs[...]
    s = sim1[...]
    t = _st(olen1, v)
    en = v[5:6] + s * v[3:4] + t * v[4:5]
    sig_o[...] = jax.nn.sigmoid(en)
    e1f = s * v[0:1] + t * v[1:2] + v[2:3] + jnp.maximum(en, 0.0)
    e0_o[...] = e1f[:, :H]
    e1_o[...] = e1f[:, H:]


def _mid_body(g0, g1, ge, e0, e1, wc, pk0_o, pk1_o, e0_o, e1_o):
    e = jnp.concatenate([e0[...], e1[...]], axis=1)
    ce = _mm(e, wc[...])
    en0 = g0[:, :H] + ge[:, :H] + ce[:, :H]
    sg0 = jax.nn.sigmoid(en0)
    pk0_o[...] = jnp.concatenate([sg0 * g0[:, H:], sg0], axis=1)
    en1 = g1[:, :H] + ge[:, H:] + ce[:, H:]
    sg1 = jax.nn.sigmoid(en1)
    pk1_o[...] = jnp.concatenate([sg1 * g1[:, H:], sg1], axis=1)
    e0_o[...] = e0[...] + jnp.maximum(en0, 0.0)
    e1_o[...] = e1[...] + jnp.maximum(en1, 0.0)


def _dectc_body(gs, gd, sim1, olen1, vecs, p_o):
    v = vecs[...]
    s = sim1[...]
    t = _st(olen1, v)
    z = gs[...] + gd[...] + s * v[10:11] + t * v[11:12] + v[12:13]
    p_o[...] = (jnp.sum(jnp.maximum(z, 0.0) * v[13:14], axis=1, keepdims=True)
                + v[14:15, 0:1])


def _espec(width):
    return pl.BlockSpec((_EBLK, width), lambda i: (i, 0))


def _wspec(rows=D):
    return pl.BlockSpec((rows, D), lambda i: (0, 0))


def _l0tc(sim1, olen1, vecs):
    return pl.pallas_call(
        _l0tc_body,
        grid=(E // _EBLK,),
        in_specs=[_espec(1), _espec(1), _wspec(16)],
        out_specs=[_espec(D), _espec(H), _espec(H)],
        out_shape=[jax.ShapeDtypeStruct((E, D), jnp.float32),
                   jax.ShapeDtypeStruct((E, H), jnp.float32),
                   jax.ShapeDtypeStruct((E, H), jnp.float32)],
    )(sim1, olen1, vecs)


def _mid(g0, g1, ge, e0, e1, wc):
    return pl.pallas_call(
        _mid_body,
        grid=(E // _EBLK,),
        in_specs=[_espec(D), _espec(D), _espec(D), _espec(H), _espec(H),
                  _wspec()],
        out_specs=[_espec(D), _espec(D), _espec(H), _espec(H)],
        out_shape=[jax.ShapeDtypeStruct((E, D), jnp.float32),
                   jax.ShapeDtypeStruct((E, D), jnp.float32),
                   jax.ShapeDtypeStruct((E, H), jnp.float32),
                   jax.ShapeDtypeStruct((E, H), jnp.float32)],
    )(g0, g1, ge, e0, e1, wc)


def _dectc(gs, gd, sim1, olen1, vecs):
    return pl.pallas_call(
        _dectc_body,
        grid=(E // _EBLK,),
        in_specs=[_espec(D), _espec(D), _espec(1), _espec(1), _wspec(16)],
        out_specs=[_espec(1)],
        out_shape=[jax.ShapeDtypeStruct((E, 1), jnp.float32)],
    )(gs, gd, sim1, olen1, vecs)


def _tables_out(h, wa, wb, wd, we, outs):
    (h_o, ah_o, db0, db1, ehf) = outs
    ah = _mm(h, wa[...])
    bh = _mm(h, wb[...])
    dh = _mm(h, wd[...])
    eh = _mm(h, we[...])
    h_o[...] = h
    ah_o[...] = ah
    db0[...] = jnp.concatenate([dh[:, :H], bh[:, :H]], axis=1)
    db1[...] = jnp.concatenate([dh[:, H:], bh[:, H:]], axis=1)
    ehf[...] = eh


def _hupd(hp, ahp, p0, p1):
    r0 = (p0[0, :, :H] + p0[1, :, :H]) / (p0[0, :, H:] + p0[1, :, H:] + 1e-6)
    r1 = (p1[0, :, :H] + p1[1, :, :H]) / (p1[0, :, H:] + p1[1, :, H:] + 1e-6)
    return hp[...] + jnp.maximum(
        ahp[...] + jnp.concatenate([r0, r1], axis=1), 0.0)


def _node1_body(sp, vecs, wa, wb, wd, we, *outs):
    s = sp[0] + sp[1]
    a0 = vecs[8:9, :]
    b0 = vecs[9:10, :]
    h = 1.0 + jnp.maximum(a0 + (b0 * s) / (s + 1e-6), 0.0)
    _tables_out(h, wa, wb, wd, we, outs)


def _node_body(hp, ahp, p0, p1, wa, wb, wd, we, *outs):
    h = _hupd(hp, ahp, p0, p1)
    _tables_out(h, wa, wb, wd, we, outs)


def _decnode_body(hp, ahp, p0, p1, w1, w2, hs_o, hd_o):
    h = _hupd(hp, ahp, p0, p1)
    hs_o[...] = _mm(h, w1[...])
    hd_o[...] = _mm(h, w2[...])


def _nspec(width):
    return pl.BlockSpec((_NBLK, width), lambda i: (i, 0))


def _pspec(width):
    return pl.BlockSpec((NC, _NBLK, width), lambda i: (0, i, 0))


_TABLE_OUT_SHAPES = [jax.ShapeDtypeStruct((N, D), jnp.float32)] * 5
_TABLE_OUT_SPECS = [_nspec(D)] * 5


def _node1(sp, vecs, wa, wb, wd, we):
    return pl.pallas_call(
        _node1_body,
        grid=(N // _NBLK,),
        in_specs=[_pspec(D), _wspec(16), _wspec(), _wspec(), _wspec(),
                  _wspec()],
        out_specs=_TABLE_OUT_SPECS,
        out_shape=_TABLE_OUT_SHAPES,
    )(sp, vecs, wa, wb, wd, we)


def _node(hp, ahp, p0, p1, wa, wb, wd, we):
    return pl.pallas_call(
        _node_body,
        grid=(N // _NBLK,),
        in_specs=[_nspec(D), _nspec(D), _pspec(D), _pspec(D),
                  _wspec(), _wspec(), _wspec(), _wspec()],
        out_specs=_TABLE_OUT_SPECS,
        out_shape=_TABLE_OUT_SHAPES,
    )(hp, ahp, p0, p1, wa, wb, wd, we)


def _decnode(hp, ahp, p0, p1, w1, w2):
    return pl.pallas_call(
        _decnode_body,
        grid=(N // _NBLK,),
        in_specs=[_nspec(D), _nspec(D), _pspec(D), _pspec(D),
                  _wspec(), _wspec()],
        out_specs=[_nspec(D), _nspec(D)],
        out_shape=[jax.ShapeDtypeStruct((N, D), jnp.float32),
                   jax.ShapeDtypeStruct((N, D), jnp.float32)],
    )(hp, ahp, p0, p1, w1, w2)


def kernel(edge_index, overlap_similarity, overlap_length, W_enc, b_enc,
           Wa, Wb, Wc, Wd, We, Wd1, bd1, Wd2, bd2):
    src = edge_index[0]
    dst = edge_index[1]
    sim1 = overlap_similarity.reshape(E, 1)
    olen1 = overlap_length.reshape(E, 1)

    vecs = _prep(overlap_length, W_enc, b_enc, Wa[0], Wb[0], Wd[0], We[0],
                 Wc[0], Wd1, bd1, Wd2, bd2)

    sig, e0h, e1h = _l0tc(sim1, olen1, vecs)
    (sp,) = _scat(dst, sig)

    node_in = _node1(sp, vecs, Wa[1], Wb[1], Wd[1], We[1])
    for l in (1, 2, 3):
        h, ah, db0, db1, ehf = node_in
        g0, g1, ge = _gather3(src, dst, db0, db1, ehf)
        pk0, pk1, e0h, e1h = _mid(g0, g1, ge, e0h, e1h, Wc[l])
        (p0,) = _scat(dst, pk0)
        (p1,) = _scat(dst, pk1)
        if l < 3:
            node_in = _node(h, ah, p0, p1,
                            Wa[l + 1], Wb[l + 1], Wd[l + 1], We[l + 1])

    hs, hd = _decnode(h, ah, p0, p1, Wd1[:D], Wd1[D:2 * D])
    gs, gd = _gather2(src, dst, hs, hd)
    (p,) = _dectc(gs, gd, sim1, olen1, vecs)
    return p
